# baseline (speedup 1.0000x reference)
import functools

import numpy as np
import jax
import jax.numpy as jnp
from jax import lax
from jax.experimental import pallas as pl
from jax.experimental.pallas import tpu as pltpu
from jax.experimental.pallas import tpu_sc as plsc

_MAX_PARAMS = 524288
_LEVELS = 16
_BASE_RES = 16.0
_MAX_RES = 2048.0
_FEAT = 2
_B = 131072

_P2 = -1640531535
_P3 = 805459861


def _layout():
    log_b = np.log(_MAX_RES / _BASE_RES) / (_LEVELS - 1)
    offs, sizes, scales = [], [], []
    off = 0
    for i in range(_LEVELS):
        res = np.ceil(_BASE_RES * np.exp(i * log_b) - 1.0) + 1.0
        aligned = int((res ** 3 + 7) // 8) * 8
        sz = int(min(_MAX_PARAMS, aligned))
        offs.append(off)
        sizes.append(sz)
        scales.append(float(_BASE_RES * np.exp(i * log_b) - 1.0))
        off += sz
    return offs, sizes, scales


_OFFS, _SIZES, _SCALES = _layout()

_NC, _NS = 2, 16
_NW = _NC * _NS
_PTS = _B // _NW
_S = 1024
_NSUB = _PTS // _S
_G = _S // 16


@functools.cache
def _build():
  mesh = plsc.VectorSubcoreMesh(core_axis_name="c", subcore_axis_name="s")

  @functools.partial(
      pl.kernel,
      out_type=jax.ShapeDtypeStruct((2 * _LEVELS, _B), jnp.float32),
      mesh=mesh,
      scratch_types=[
          pltpu.VMEM((3, _S), jnp.float32),
          pltpu.VMEM((16 * _S,), jnp.int32),
          pltpu.VMEM((16 * _S,), jnp.int32),
          pltpu.VMEM((8 * _S * 2,), jnp.float32),
          pltpu.VMEM((8 * _S * 2,), jnp.float32),
          pltpu.VMEM((8, _S), jnp.float32),
          pltpu.VMEM((8, _S), jnp.float32),
          pltpu.VMEM((2 * _LEVELS, _S), jnp.float32),
          pltpu.SemaphoreType.DMA,
          pltpu.SemaphoreType.DMA,
      ],
  )
  def _hash_enc(xyz_t, table, out, xyz_v, idx0, idx1, rows0, rows1, w0, w1,
                ob, sem0, sem1):
    wid = lax.axis_index("s") * _NC + lax.axis_index("c")
    tile_base = wid * _PTS
    idxb = (idx0, idx1)
    rowsb = (rows0, rows1)
    wb = (w0, w1)
    sems = (sem0, sem1)

    iota = lax.iota(jnp.int32, 16)
    zero_i = jnp.zeros((16,), jnp.int32)
    one_i = jnp.full((16,), 1, jnp.int32)

    def umod(h, size):
      if size & (size - 1) == 0:
        return h & (size - 1)
      lo = h & 0x7FFFFFFF
      r = lax.rem(lo, jnp.full((16,), size, jnp.int32))
      c1 = (1 << 31) % size
      r = r + jnp.where(h < 0, jnp.full((16,), c1, jnp.int32), zero_i)
      return lax.rem(r, jnp.full((16,), size, jnp.int32))

    def compute_group(lvl, g, idx_r, w_r):
      scale = _SCALES[lvl]
      size = _SIZES[lvl]
      off = _OFFS[lvl]
      base16 = g * 16
      px = xyz_v[0, pl.ds(base16, 16)] * scale + 0.5
      py = xyz_v[1, pl.ds(base16, 16)] * scale + 0.5
      pz = xyz_v[2, pl.ds(base16, 16)] * scale + 0.5
      ix = px.astype(jnp.int32)
      iy = py.astype(jnp.int32)
      iz = pz.astype(jnp.int32)
      fx = px - ix.astype(jnp.float32)
      fy = py - iy.astype(jnp.float32)
      fz = pz - iz.astype(jnp.float32)
      hx = (ix, ix + 1)
      hy = (iy * _P2, (iy + 1) * _P2)
      hz = (iz * _P3, (iz + 1) * _P3)
      wx = (1.0 - fx, fx)
      wy = (1.0 - fy, fy)
      wz = (1.0 - fz, fz)
      for c in range(8):
        dx, dy, dz = (c >> 2) & 1, (c >> 1) & 1, c & 1
        h = hx[dx] ^ hy[dy] ^ hz[dz]
        idx2 = (umod(h, size) + off) * 2
        idx_r[pl.ds(c * _S + base16, 16)] = idx2
        idx_r[pl.ds(8 * _S + c * _S + base16, 16)] = idx2 + 1
        w_r[c, pl.ds(base16, 16)] = (wx[dx] * wy[dy]) * wz[dz]

    def apply_group(lvl, g, w_r, rows_r):
      base16 = g * 16
      acc0 = jnp.zeros((16,), jnp.float32)
      acc1 = jnp.zeros((16,), jnp.float32)
      for c in range(8):
        f0 = rows_r[pl.ds(c * _S + base16, 16)]
        f1 = rows_r[pl.ds(8 * _S + c * _S + base16, 16)]
        w = w_r[c, pl.ds(base16, 16)]
        acc0 = acc0 + w * f0
        acc1 = acc1 + w * f1
      ob[2 * lvl, pl.ds(base16, 16)] = acc0
      ob[2 * lvl + 1, pl.ds(base16, 16)] = acc1

    def launch_level(lvl):
      b = lvl & 1

      def gbody(g, carry):
        compute_group(lvl, g, idxb[b], wb[b])
        return carry

      lax.fori_loop(0, _G, gbody, 0)
      return pltpu.async_copy(table.at[idxb[b]], rowsb[b], sems[b])

    def apply_level(lvl):
      b = lvl & 1

      def gbody(g, carry):
        apply_group(lvl, g, wb[b], rowsb[b])
        return carry

      lax.fori_loop(0, _G, gbody, 0)

    def do_sub(s, carry):
      pbase = tile_base + s * _S
      pltpu.sync_copy(xyz_t.at[:, pl.ds(pbase, _S)], xyz_v)
      cp = launch_level(0)
      for lvl in range(1, _LEVELS):
        cp_next = launch_level(lvl)
        cp.wait()
        apply_level(lvl - 1)
        cp = cp_next
      cp.wait()
      apply_level(_LEVELS - 1)
      pltpu.sync_copy(ob, out.at[:, pl.ds(pbase, _S)])
      return carry

    lax.fori_loop(0, _NSUB, do_sub, 0)

  return _hash_enc


def kernel(xyzs, hash_table, offsets, hash_map_sizes):
    del offsets, hash_map_sizes
    chan_major = _build()(xyzs.T, hash_table.reshape(-1))
    return chan_major.T.reshape(_B, _LEVELS, _FEAT)


# speedup vs baseline: 1.0005x; 1.0005x over previous
import functools

import numpy as np
import jax
import jax.numpy as jnp
from jax import lax
from jax.experimental import pallas as pl
from jax.experimental.pallas import tpu as pltpu
from jax.experimental.pallas import tpu_sc as plsc

_MAX_PARAMS = 524288
_LEVELS = 16
_BASE_RES = 16.0
_MAX_RES = 2048.0
_FEAT = 2
_B = 131072

_P2 = -1640531535
_P3 = 805459861


def _layout():
    log_b = np.log(_MAX_RES / _BASE_RES) / (_LEVELS - 1)
    offs, sizes, scales = [], [], []
    off = 0
    for i in range(_LEVELS):
        res = np.ceil(_BASE_RES * np.exp(i * log_b) - 1.0) + 1.0
        aligned = int((res ** 3 + 7) // 8) * 8
        sz = int(min(_MAX_PARAMS, aligned))
        offs.append(off)
        sizes.append(sz)
        scales.append(float(_BASE_RES * np.exp(i * log_b) - 1.0))
        off += sz
    return offs, sizes, scales


_OFFS, _SIZES, _SCALES = _layout()

_NC, _NS = 2, 16
_NW = _NC * _NS
_PTS = _B // _NW
_S = 1024
_NSUB = _PTS // _S
_G = _S // 16


@functools.cache
def _build():
  mesh = plsc.VectorSubcoreMesh(core_axis_name="c", subcore_axis_name="s")

  @functools.partial(
      pl.kernel,
      out_type=jax.ShapeDtypeStruct((2 * _LEVELS, _B), jnp.float32),
      mesh=mesh,
      compiler_params=pltpu.CompilerParams(needs_layout_passes=False),
      scratch_types=[
          pltpu.VMEM((3, _S), jnp.float32),
          pltpu.VMEM((16 * _S,), jnp.int32),
          pltpu.VMEM((16 * _S,), jnp.int32),
          pltpu.VMEM((16 * _S,), jnp.float32),
          pltpu.VMEM((16 * _S,), jnp.float32),
          pltpu.VMEM((8, _S), jnp.float32),
          pltpu.VMEM((8, _S), jnp.float32),
          pltpu.VMEM((2 * _LEVELS, _S), jnp.float32),
          pltpu.SemaphoreType.DMA,
          pltpu.SemaphoreType.DMA,
      ],
  )
  def _hash_enc(xyz_t, table, out, xyz_v, idx0, idx1, rows0, rows1, w0, w1,
                ob, sem0, sem1):
    wid = lax.axis_index("s") * _NC + lax.axis_index("c")
    tile_base = wid * _PTS
    idxb = (idx0, idx1)
    rowsb = (rows0, rows1)
    wb = (w0, w1)
    sems = (sem0, sem1)

    iota = lax.iota(jnp.int32, 16)
    zero_i = jnp.zeros((16,), jnp.int32)
    one_i = jnp.full((16,), 1, jnp.int32)

    def umod(h, size):
      if size & (size - 1) == 0:
        return h & (size - 1)
      lo = h & 0x7FFFFFFF
      r = lax.rem(lo, jnp.full((16,), size, jnp.int32))
      c1 = (1 << 31) % size
      r = r + jnp.where(h < 0, jnp.full((16,), c1, jnp.int32), zero_i)
      return lax.rem(r, jnp.full((16,), size, jnp.int32))

    def compute_group(lvl, g, idx_r, w_r):
      scale = _SCALES[lvl]
      size = _SIZES[lvl]
      off = _OFFS[lvl]
      base16 = g * 16
      px = xyz_v[0, pl.ds(base16, 16)] * scale + 0.5
      py = xyz_v[1, pl.ds(base16, 16)] * scale + 0.5
      pz = xyz_v[2, pl.ds(base16, 16)] * scale + 0.5
      ix = px.astype(jnp.int32)
      iy = py.astype(jnp.int32)
      iz = pz.astype(jnp.int32)
      fx = px - ix.astype(jnp.float32)
      fy = py - iy.astype(jnp.float32)
      fz = pz - iz.astype(jnp.float32)
      hx = (ix, ix + 1)
      hy = (iy * _P2, (iy + 1) * _P2)
      hz = (iz * _P3, (iz + 1) * _P3)
      wx = (1.0 - fx, fx)
      wy = (1.0 - fy, fy)
      wz = (1.0 - fz, fz)
      for c in range(8):
        dx, dy, dz = (c >> 2) & 1, (c >> 1) & 1, c & 1
        h = hx[dx] ^ hy[dy] ^ hz[dz]
        idx2 = (umod(h, size) + off) * 2
        idx_r[pl.ds(c * _S + base16, 16)] = idx2
        idx_r[pl.ds(8 * _S + c * _S + base16, 16)] = idx2 + 1
        w_r[c, pl.ds(base16, 16)] = (wx[dx] * wy[dy]) * wz[dz]

    def apply_group(lvl, g, w_r, rows_r):
      base16 = g * 16
      acc0 = jnp.zeros((16,), jnp.float32)
      acc1 = jnp.zeros((16,), jnp.float32)
      for c in range(8):
        f0 = rows_r[pl.ds(c * _S + base16, 16)]
        f1 = rows_r[pl.ds(8 * _S + c * _S + base16, 16)]
        w = w_r[c, pl.ds(base16, 16)]
        acc0 = acc0 + w * f0
        acc1 = acc1 + w * f1
      ob[2 * lvl, pl.ds(base16, 16)] = acc0
      ob[2 * lvl + 1, pl.ds(base16, 16)] = acc1

    def launch_level(lvl):
      b = lvl & 1

      def gbody(g, carry):
        compute_group(lvl, g, idxb[b], wb[b])
        return carry

      lax.fori_loop(0, _G, gbody, 0)
      return pltpu.async_copy(table.at[idxb[b]], rowsb[b], sems[b])

    def apply_level(lvl):
      b = lvl & 1

      def gbody(g, carry):
        apply_group(lvl, g, wb[b], rowsb[b])
        return carry

      lax.fori_loop(0, _G, gbody, 0)

    def do_sub(s, carry):
      pbase = tile_base + s * _S
      pltpu.sync_copy(xyz_t.at[:, pl.ds(pbase, _S)], xyz_v)
      cp = launch_level(0)
      for lvl in range(1, _LEVELS):
        cp_next = launch_level(lvl)
        cp.wait()
        apply_level(lvl - 1)
        cp = cp_next
      cp.wait()
      apply_level(_LEVELS - 1)
      pltpu.sync_copy(ob, out.at[:, pl.ds(pbase, _S)])
      return carry

    lax.fori_loop(0, _NSUB, do_sub, 0)

  return _hash_enc


def kernel(xyzs, hash_table, offsets, hash_map_sizes):
    del offsets, hash_map_sizes
    chan_major = _build()(xyzs.T, hash_table.reshape(-1))
    return chan_major.T.reshape(_B, _LEVELS, _FEAT)
